# baseline (speedup 1.0000x reference)
import functools

import jax
import jax.numpy as jnp
from jax import lax
from jax.experimental import pallas as pl
from jax.experimental.pallas import tpu as pltpu
from jax.experimental.pallas import tpu_sc as plsc

_NW = 32
_CHUNK = 128
_NBUF = 2


def _build_gather(nchunk, c, d, dtype):
  mesh = plsc.VectorSubcoreMesh(core_axis_name="c", subcore_axis_name="s")
  b_per_w = nchunk * c

  @functools.partial(
      pl.kernel,
      out_type=jax.ShapeDtypeStruct((_NW * b_per_w, d), dtype),
      mesh=mesh,
      scratch_types=[
          pltpu.VMEM((nchunk, c), jnp.int32),
          pltpu.VMEM((c, d), dtype),
          pltpu.VMEM((c, d), dtype),
          pltpu.SemaphoreType.DMA,
          pltpu.SemaphoreType.DMA,
      ],
      compiler_params=pltpu.CompilerParams(use_tc_tiling_on_sc=False),
  )
  def k(idx_hbm, table_hbm, out_hbm, idx_v, buf0, buf1, sem0, sem1):
    wid = lax.axis_index("s") * 2 + lax.axis_index("c")
    base = wid * b_per_w
    pltpu.sync_copy(idx_hbm.at[wid], idx_v)
    bufs = (buf0, buf1)
    sems = (sem0, sem1)
    for b in range(_NBUF):
      pltpu.async_copy(table_hbm.at[idx_v.at[b]], bufs[b], sems[b])

    def body(g, carry):
      for b in range(_NBUF):
        j = g * _NBUF + b
        pltpu.make_async_copy(table_hbm.at[idx_v.at[j]], bufs[b], sems[b]).wait()
        pltpu.sync_copy(bufs[b], out_hbm.at[pl.ds(base + j * c, c)])
        pltpu.async_copy(table_hbm.at[idx_v.at[j + _NBUF]], bufs[b], sems[b])
      return carry

    steady = (nchunk - _NBUF) // _NBUF
    lax.fori_loop(0, steady, body, 0)
    for b in range(_NBUF):
      j = steady * _NBUF + b
      pltpu.make_async_copy(table_hbm.at[idx_v.at[j]], bufs[b], sems[b]).wait()
      pltpu.sync_copy(bufs[b], out_hbm.at[pl.ds(base + j * c, c)])

  return k


def kernel(inputs, embeddings):
  bsz, seq = inputs.shape
  d = embeddings.shape[1]
  total = bsz * seq
  nchunk = total // (_NW * _CHUNK)
  idx3 = inputs.astype(jnp.int32).reshape(_NW, nchunk, _CHUNK)
  out = _build_gather(nchunk, _CHUNK, d, embeddings.dtype)(idx3, embeddings)
  return out.reshape(bsz, seq, d)


# speedup vs baseline: 1.0090x; 1.0090x over previous
import functools

import jax
import jax.numpy as jnp
from jax import lax
from jax.experimental import pallas as pl
from jax.experimental.pallas import tpu as pltpu
from jax.experimental.pallas import tpu_sc as plsc

_NW = 32
_CHUNK = 128
_NBUF = 5


def _build_gather(nchunk, c, d, dtype):
  mesh = plsc.VectorSubcoreMesh(core_axis_name="c", subcore_axis_name="s")
  b_per_w = nchunk * c

  @functools.partial(
      pl.kernel,
      out_type=jax.ShapeDtypeStruct((_NW * b_per_w, d), dtype),
      mesh=mesh,
      scratch_types=[
          pltpu.VMEM((nchunk, c), jnp.int32),
      ] + [pltpu.VMEM((c, d), dtype) for _ in range(_NBUF)]
        + [pltpu.SemaphoreType.DMA for _ in range(_NBUF)],
      compiler_params=pltpu.CompilerParams(use_tc_tiling_on_sc=False),
  )
  def k(idx_hbm, table_hbm, out_hbm, idx_v, *scratch):
    bufs = scratch[:_NBUF]
    sems = scratch[_NBUF:]
    wid = lax.axis_index("s") * 2 + lax.axis_index("c")
    base = wid * b_per_w
    pltpu.sync_copy(idx_hbm.at[wid], idx_v)
    for b in range(_NBUF):
      pltpu.async_copy(table_hbm.at[idx_v.at[b]], bufs[b], sems[b])

    def body(g, carry):
      for b in range(_NBUF):
        j = g * _NBUF + b
        pltpu.make_async_copy(table_hbm.at[idx_v.at[j]], bufs[b], sems[b]).wait()
        pltpu.sync_copy(bufs[b], out_hbm.at[pl.ds(base + j * c, c)])
        pltpu.async_copy(table_hbm.at[idx_v.at[j + _NBUF]], bufs[b], sems[b])
      return carry

    steady = (nchunk - _NBUF) // _NBUF
    lax.fori_loop(0, steady, body, 0)
    for b in range(_NBUF):
      j = steady * _NBUF + b
      pltpu.make_async_copy(table_hbm.at[idx_v.at[j]], bufs[b], sems[b]).wait()
      pltpu.sync_copy(bufs[b], out_hbm.at[pl.ds(base + j * c, c)])

  return k


def kernel(inputs, embeddings):
  bsz, seq = inputs.shape
  d = embeddings.shape[1]
  total = bsz * seq
  nchunk = total // (_NW * _CHUNK)
  idx3 = inputs.astype(jnp.int32).reshape(_NW, nchunk, _CHUNK)
  out = _build_gather(nchunk, _CHUNK, d, embeddings.dtype)(idx3, embeddings)
  return out.reshape(bsz, seq, d)
